# baseline (speedup 1.0000x reference)
import functools

import jax
import jax.numpy as jnp
from jax import lax
from jax.experimental import pallas as pl
from jax.experimental.pallas import tpu as pltpu
from jax.experimental.pallas import tpu_sc as plsc

N = 10000
E = 320000
D = 128
C = 40

NC = 2
NS = 16
NW = NC * NS
EW = E // NW
K = 200
EWP = 10000
EPAD = NW * EWP
NCHUNK = EWP // K
EPW = EWP - EW
NA = N + NS * 8
RW = 624
RTAIL = N - NS * RW
KD = 2000
NDCHUNK = EW // KD

BL = 2000
GRID = N // BL


def _deg_body(col_hbm, z_hbm, o_hbm, out_hbm, col_v, ones_v, deg_v, deg_sh):
    c = lax.axis_index("c")
    s = lax.axis_index("s")

    @pl.when(s == 0)
    def _():
        pltpu.sync_copy(z_hbm, deg_v)
        pltpu.sync_copy(deg_v, deg_sh)
    pltpu.sync_copy(o_hbm, ones_v)
    plsc.subcore_barrier()

    ebase = (c * NS + s) * EW

    def body(i, carry):
        off = ebase + i * KD
        pltpu.sync_copy(col_hbm.at[pl.ds(off, KD)], col_v)
        pltpu.sync_copy(ones_v, deg_sh.at[col_v], add=True)
        return carry

    lax.fori_loop(0, NDCHUNK, body, 0)
    plsc.subcore_barrier()

    @pl.when(s == 0)
    def _():
        pltpu.sync_copy(deg_sh, deg_v)
        pltpu.sync_copy(deg_v, out_hbm.at[pl.ds(c * N, N)])


def _make_deg_kernel():
    mesh = plsc.VectorSubcoreMesh(core_axis_name="c", subcore_axis_name="s")
    return pl.kernel(
        _deg_body,
        out_type=jax.ShapeDtypeStruct((NC * N,), jnp.float32),
        mesh=mesh,
        scratch_types=[
            pltpu.VMEM((KD,), jnp.int32),
            pltpu.VMEM((KD,), jnp.float32),
            pltpu.VMEM((N,), jnp.float32),
            pltpu.VMEM_SHARED((N,), jnp.float32),
        ],
    )


def _scatter_body(g_hbm, row_hbm, col_hbm, z_hbm, out_hbm,
                  ir0, ic0, r0, gs0, acc_sh):
    c = lax.axis_index("c")
    s = lax.axis_index("s")
    w = c * NS + s
    nbase = s * RW

    pltpu.sync_copy(z_hbm.at[pl.ds(nbase, RW)], acc_sh.at[pl.ds(nbase, RW)])

    @pl.when(s == 0)
    def _():
        pltpu.sync_copy(z_hbm.at[pl.ds(NS * RW, RTAIL)],
                        acc_sh.at[pl.ds(NS * RW, RTAIL)])

    ebase = w * EWP

    def body(j, carry):
        off = ebase + j * K
        pltpu.sync_copy(row_hbm.at[pl.ds(off, K)], ir0)
        pltpu.sync_copy(col_hbm.at[pl.ds(off, K)], ic0)
        pltpu.async_copy(g_hbm.at[ir0], r0, gs0).wait()
        pltpu.sync_copy(r0, acc_sh.at[ic0], add=True)
        return carry

    lax.fori_loop(0, NCHUNK, body, 0)
    plsc.subcore_barrier()

    pltpu.sync_copy(acc_sh.at[pl.ds(nbase, RW)],
                    out_hbm.at[c, pl.ds(nbase, RW)])

    @pl.when(s == 0)
    def _():
        pltpu.sync_copy(acc_sh.at[pl.ds(NS * RW, RTAIL)],
                        out_hbm.at[c, pl.ds(NS * RW, RTAIL)])


def _make_scatter_kernel():
    mesh = plsc.VectorSubcoreMesh(core_axis_name="c", subcore_axis_name="s")
    return pl.kernel(
        _scatter_body,
        out_type=jax.ShapeDtypeStruct((NC, N, D), jnp.float32),
        mesh=mesh,
        scratch_types=[
            pltpu.VMEM((K,), jnp.int32),
            pltpu.VMEM((K,), jnp.int32),
            pltpu.VMEM((K, D), jnp.float32),
            pltpu.SemaphoreType.DMA,
            pltpu.VMEM_SHARED((NA, D), jnp.float32),
        ],
    )


def _dotT(a, w):
    return lax.dot_general(a, w, (((1,), (1,)), ((), ())),
                           preferred_element_type=jnp.float32)


def _pre_body(x_ref, w_ref, b_ref, deg_ref, h_ref, g_ref, dis_ref):
    degt = deg_ref[0] + deg_ref[1]
    dis = jnp.where(degt > 0.0, lax.rsqrt(jnp.maximum(degt, 1e-30)), 0.0)
    h = _dotT(x_ref[...], w_ref[...]) + b_ref[...]
    h_ref[...] = h
    g_ref[...] = h * dis
    dis_ref[...] = dis


def _conv_body(s_ref, h_ref, dis_ref, wa_ref, wb_ref, h2_ref, g2_ref):
    dis = dis_ref[...]
    hi = (s_ref[0] + s_ref[1]) * dis
    t = _dotT(hi, wa_ref[...]) + _dotT(h_ref[...], wb_ref[...])
    h2 = jax.nn.gelu(t)
    h2_ref[...] = h2
    g2_ref[...] = h2 * dis


def _final_body(s_ref, h_ref, dis_ref, wa_ref, wb_ref, wc_ref, bc_ref,
                out_ref):
    dis = dis_ref[...]
    hi = (s_ref[0] + s_ref[1]) * dis
    t = _dotT(hi, wa_ref[...]) + _dotT(h_ref[...], wb_ref[...])
    h3 = jax.nn.gelu(t)
    out_ref[...] = _dotT(h3, wc_ref[...]) + bc_ref[...]


_row_spec = pl.BlockSpec((BL, D), lambda i: (i, 0))
_dis_spec = pl.BlockSpec((BL, 1), lambda i: (i, 0))
_s_spec = pl.BlockSpec((NC, BL, D), lambda i: (0, i, 0))
_w_spec = pl.BlockSpec((D, D), lambda i: (0, 0))


def _pre_call(x, w_pre, b_pre2, deg):
    return pl.pallas_call(
        _pre_body,
        grid=(GRID,),
        in_specs=[
            _row_spec,
            _w_spec,
            pl.BlockSpec((1, D), lambda i: (0, 0)),
            pl.BlockSpec((NC, BL, 1), lambda i: (0, i, 0)),
        ],
        out_specs=[_row_spec, _row_spec, _dis_spec],
        out_shape=[
            jax.ShapeDtypeStruct((N, D), jnp.float32),
            jax.ShapeDtypeStruct((N, D), jnp.float32),
            jax.ShapeDtypeStruct((N, 1), jnp.float32),
        ],
    )(x, w_pre, b_pre2, deg)


def _conv_call(s, h, dis, wa, wb):
    return pl.pallas_call(
        _conv_body,
        grid=(GRID,),
        in_specs=[_s_spec, _row_spec, _dis_spec, _w_spec, _w_spec],
        out_specs=[_row_spec, _row_spec],
        out_shape=[
            jax.ShapeDtypeStruct((N, D), jnp.float32),
            jax.ShapeDtypeStruct((N, D), jnp.float32),
        ],
    )(s, h, dis, wa, wb)


def _final_call(s, h, dis, wa, wb, w_cls, b_cls2):
    return pl.pallas_call(
        _final_body,
        grid=(GRID,),
        in_specs=[
            _s_spec, _row_spec, _dis_spec, _w_spec, _w_spec,
            pl.BlockSpec((C, D), lambda i: (0, 0)),
            pl.BlockSpec((1, C), lambda i: (0, 0)),
        ],
        out_specs=pl.BlockSpec((BL, C), lambda i: (i, 0)),
        out_shape=jax.ShapeDtypeStruct((N, C), jnp.float32),
    )(s, h, dis, wa, wb, w_cls, b_cls2)


def kernel(x, edge_index, W_pre, b_pre, W1, W2, W_cls, b_cls):
    row = edge_index[0]
    col = edge_index[1]

    zeros_big = jnp.zeros((N, D), jnp.float32)
    zeros_deg = jnp.zeros((N,), jnp.float32)
    ones_deg = jnp.ones((KD,), jnp.float32)

    deg = _make_deg_kernel()(col, zeros_deg, ones_deg).reshape(NC, N, 1)

    b_pre2 = b_pre.reshape(1, D)
    h, g1, dis = _pre_call(x, W_pre, b_pre2, deg)


    scatter = _make_scatter_kernel()
    s1 = scatter(g1, row, col, zeros_big)

    h2, g2 = _conv_call(s1, h, dis, W1[:, :D], W1[:, D:])

    s2 = scatter(g2, row, col, zeros_big)

    logits = _final_call(s2, h2, dis, W2[:, :D], W2[:, D:],
                         W_cls, b_cls.reshape(1, C))
    return logits


# speedup vs baseline: 1.2294x; 1.2294x over previous
import functools

import jax
import jax.numpy as jnp
from jax import lax
from jax.experimental import pallas as pl
from jax.experimental.pallas import tpu as pltpu
from jax.experimental.pallas import tpu_sc as plsc

N = 10000
E = 320000
D = 128
C = 40

NC = 2
NS = 16
NW = NC * NS
EW = E // NW
K = 200
EWP = 10000
EPAD = NW * EWP
NCHUNK = EWP // K
EPW = EWP - EW
NA = N + NS * 8
RW = 624
RTAIL = N - NS * RW
KD = 2000
NDCHUNK = EW // KD

BL = 2000
GRID = N // BL


def _deg_body(col_hbm, z_hbm, o_hbm, out_hbm, col_v, ones_v, deg_v, deg_sh):
    c = lax.axis_index("c")
    s = lax.axis_index("s")

    @pl.when(s == 0)
    def _():
        pltpu.sync_copy(z_hbm, deg_v)
        pltpu.sync_copy(deg_v, deg_sh)
    pltpu.sync_copy(o_hbm, ones_v)
    plsc.subcore_barrier()

    ebase = (c * NS + s) * EW

    def body(i, carry):
        off = ebase + i * KD
        pltpu.sync_copy(col_hbm.at[pl.ds(off, KD)], col_v)
        pltpu.sync_copy(ones_v, deg_sh.at[col_v], add=True)
        return carry

    lax.fori_loop(0, NDCHUNK, body, 0)
    plsc.subcore_barrier()

    @pl.when(s == 0)
    def _():
        pltpu.sync_copy(deg_sh, deg_v)
        pltpu.sync_copy(deg_v, out_hbm.at[pl.ds(c * N, N)])


def _make_deg_kernel():
    mesh = plsc.VectorSubcoreMesh(core_axis_name="c", subcore_axis_name="s")
    return pl.kernel(
        _deg_body,
        out_type=jax.ShapeDtypeStruct((NC * N,), jnp.float32),
        mesh=mesh,
        scratch_types=[
            pltpu.VMEM((KD,), jnp.int32),
            pltpu.VMEM((KD,), jnp.float32),
            pltpu.VMEM((N,), jnp.float32),
            pltpu.VMEM_SHARED((N,), jnp.float32),
        ],
    )


def _scatter_body(g_hbm, row_hbm, col_hbm, z_hbm, out_hbm,
                  ir0, ic0, r0, gs0, ir1, ic1, is0, is1, acc_sh):
    c = lax.axis_index("c")
    s = lax.axis_index("s")
    w = c * NS + s
    nbase = s * RW

    pltpu.sync_copy(z_hbm.at[pl.ds(nbase, RW)], acc_sh.at[pl.ds(nbase, RW)])

    @pl.when(s == 0)
    def _():
        pltpu.sync_copy(z_hbm.at[pl.ds(NS * RW, RTAIL)],
                        acc_sh.at[pl.ds(NS * RW, RTAIL)])

    ebase = w * EWP

    def idx_load(j, ir, ic, isem):
        off = ebase + j * K
        pltpu.async_copy(row_hbm.at[pl.ds(off, K)], ir, isem)
        pltpu.async_copy(col_hbm.at[pl.ds(off, K)], ic, isem)

    def idx_wait(j, ir, ic, isem):
        off = ebase + j * K
        pltpu.make_async_copy(row_hbm.at[pl.ds(off, K)], ir, isem).wait()
        pltpu.make_async_copy(col_hbm.at[pl.ds(off, K)], ic, isem).wait()

    def work(ir, ic):
        pltpu.async_copy(g_hbm.at[ir], r0, gs0).wait()
        pltpu.sync_copy(r0, acc_sh.at[ic], add=True)

    idx_load(0, ir0, ic0, is0)

    def pair(p, carry):
        j0 = 2 * p
        idx_wait(j0, ir0, ic0, is0)
        idx_load(j0 + 1, ir1, ic1, is1)
        work(ir0, ic0)
        idx_wait(j0 + 1, ir1, ic1, is1)
        idx_load(j0 + 2, ir0, ic0, is0)
        work(ir1, ic1)
        return carry

    lax.fori_loop(0, NCHUNK // 2 - 1, pair, 0)

    idx_wait(NCHUNK - 2, ir0, ic0, is0)
    idx_load(NCHUNK - 1, ir1, ic1, is1)
    work(ir0, ic0)
    idx_wait(NCHUNK - 1, ir1, ic1, is1)
    work(ir1, ic1)
    plsc.subcore_barrier()

    pltpu.sync_copy(acc_sh.at[pl.ds(nbase, RW)],
                    out_hbm.at[c, pl.ds(nbase, RW)])

    @pl.when(s == 0)
    def _():
        pltpu.sync_copy(acc_sh.at[pl.ds(NS * RW, RTAIL)],
                        out_hbm.at[c, pl.ds(NS * RW, RTAIL)])


def _make_scatter_kernel():
    mesh = plsc.VectorSubcoreMesh(core_axis_name="c", subcore_axis_name="s")
    return pl.kernel(
        _scatter_body,
        out_type=jax.ShapeDtypeStruct((NC, N, D), jnp.float32),
        mesh=mesh,
        scratch_types=[
            pltpu.VMEM((K,), jnp.int32),
            pltpu.VMEM((K,), jnp.int32),
            pltpu.VMEM((K, D), jnp.float32),
            pltpu.SemaphoreType.DMA,
            pltpu.VMEM((K,), jnp.int32),
            pltpu.VMEM((K,), jnp.int32),
            pltpu.SemaphoreType.DMA,
            pltpu.SemaphoreType.DMA,
            pltpu.VMEM_SHARED((NA, D), jnp.float32),
        ],
    )


def _dotT(a, w):
    return lax.dot_general(a, w, (((1,), (1,)), ((), ())),
                           preferred_element_type=jnp.float32)


def _pre_body(x_ref, w_ref, b_ref, deg_ref, h_ref, g_ref, dis_ref):
    degt = deg_ref[0] + deg_ref[1]
    dis = jnp.where(degt > 0.0, lax.rsqrt(jnp.maximum(degt, 1e-30)), 0.0)
    h = _dotT(x_ref[...], w_ref[...]) + b_ref[...]
    h_ref[...] = h
    g_ref[...] = h * dis
    dis_ref[...] = dis


def _conv_body(s_ref, h_ref, dis_ref, wa_ref, wb_ref, h2_ref, g2_ref):
    dis = dis_ref[...]
    hi = (s_ref[0] + s_ref[1]) * dis
    t = _dotT(hi, wa_ref[...]) + _dotT(h_ref[...], wb_ref[...])
    h2 = jax.nn.gelu(t)
    h2_ref[...] = h2
    g2_ref[...] = h2 * dis


def _final_body(s_ref, h_ref, dis_ref, wa_ref, wb_ref, wc_ref, bc_ref,
                out_ref):
    dis = dis_ref[...]
    hi = (s_ref[0] + s_ref[1]) * dis
    t = _dotT(hi, wa_ref[...]) + _dotT(h_ref[...], wb_ref[...])
    h3 = jax.nn.gelu(t)
    out_ref[...] = _dotT(h3, wc_ref[...]) + bc_ref[...]


_row_spec = pl.BlockSpec((BL, D), lambda i: (i, 0))
_dis_spec = pl.BlockSpec((BL, 1), lambda i: (i, 0))
_s_spec = pl.BlockSpec((NC, BL, D), lambda i: (0, i, 0))
_w_spec = pl.BlockSpec((D, D), lambda i: (0, 0))


def _pre_call(x, w_pre, b_pre2, deg):
    return pl.pallas_call(
        _pre_body,
        grid=(GRID,),
        in_specs=[
            _row_spec,
            _w_spec,
            pl.BlockSpec((1, D), lambda i: (0, 0)),
            pl.BlockSpec((NC, BL, 1), lambda i: (0, i, 0)),
        ],
        out_specs=[_row_spec, _row_spec, _dis_spec],
        out_shape=[
            jax.ShapeDtypeStruct((N, D), jnp.float32),
            jax.ShapeDtypeStruct((N, D), jnp.float32),
            jax.ShapeDtypeStruct((N, 1), jnp.float32),
        ],
    )(x, w_pre, b_pre2, deg)


def _conv_call(s, h, dis, wa, wb):
    return pl.pallas_call(
        _conv_body,
        grid=(GRID,),
        in_specs=[_s_spec, _row_spec, _dis_spec, _w_spec, _w_spec],
        out_specs=[_row_spec, _row_spec],
        out_shape=[
            jax.ShapeDtypeStruct((N, D), jnp.float32),
            jax.ShapeDtypeStruct((N, D), jnp.float32),
        ],
    )(s, h, dis, wa, wb)


def _final_call(s, h, dis, wa, wb, w_cls, b_cls2):
    return pl.pallas_call(
        _final_body,
        grid=(GRID,),
        in_specs=[
            _s_spec, _row_spec, _dis_spec, _w_spec, _w_spec,
            pl.BlockSpec((C, D), lambda i: (0, 0)),
            pl.BlockSpec((1, C), lambda i: (0, 0)),
        ],
        out_specs=pl.BlockSpec((BL, C), lambda i: (i, 0)),
        out_shape=jax.ShapeDtypeStruct((N, C), jnp.float32),
    )(s, h, dis, wa, wb, w_cls, b_cls2)


def kernel(x, edge_index, W_pre, b_pre, W1, W2, W_cls, b_cls):
    row = edge_index[0]
    col = edge_index[1]

    zeros_big = jnp.zeros((N, D), jnp.float32)
    zeros_deg = jnp.zeros((N,), jnp.float32)
    ones_deg = jnp.ones((KD,), jnp.float32)

    deg = _make_deg_kernel()(col, zeros_deg, ones_deg).reshape(NC, N, 1)

    b_pre2 = b_pre.reshape(1, D)
    h, g1, dis = _pre_call(x, W_pre, b_pre2, deg)


    scatter = _make_scatter_kernel()
    s1 = scatter(g1, row, col, zeros_big)

    h2, g2 = _conv_call(s1, h, dis, W1[:, :D], W1[:, D:])

    s2 = scatter(g2, row, col, zeros_big)

    logits = _final_call(s2, h2, dis, W2[:, :D], W2[:, D:],
                         W_cls, b_cls.reshape(1, C))
    return logits


# speedup vs baseline: 1.2359x; 1.0052x over previous
import functools

import jax
import jax.numpy as jnp
from jax import lax
from jax.experimental import pallas as pl
from jax.experimental.pallas import tpu as pltpu
from jax.experimental.pallas import tpu_sc as plsc

N = 10000
E = 320000
D = 128
C = 40

NC = 2
NS = 16
NW = NC * NS
EW = E // NW
K = 200
EWP = 10000
EPAD = NW * EWP
NCHUNK = EWP // K
EPW = EWP - EW
NA = N + NS * 8
RW = 624
RTAIL = N - NS * RW
KD = 2000
NDCHUNK = EW // KD

BL = 2000
GRID = N // BL


def _deg_body(col_hbm, z_hbm, o_hbm, out_hbm, cv0, cv1, ds0, ds1,
              ones_v, deg_v, deg_sh):
    c = lax.axis_index("c")
    s = lax.axis_index("s")

    @pl.when(s == 0)
    def _():
        pltpu.sync_copy(z_hbm, deg_v)
        pltpu.sync_copy(deg_v, deg_sh)
    pltpu.sync_copy(o_hbm, ones_v)
    plsc.subcore_barrier()

    ebase = (c * NS + s) * EW

    def cload(i, cv, sem):
        pltpu.async_copy(col_hbm.at[pl.ds(ebase + i * KD, KD)], cv, sem)

    def cwait(i, cv, sem):
        pltpu.make_async_copy(col_hbm.at[pl.ds(ebase + i * KD, KD)],
                              cv, sem).wait()

    cload(0, cv0, ds0)
    for i in range(NDCHUNK):
        cv, sem = (cv0, ds0) if i % 2 == 0 else (cv1, ds1)
        nv, nsem = (cv1, ds1) if i % 2 == 0 else (cv0, ds0)
        cwait(i, cv, sem)
        if i + 1 < NDCHUNK:
            cload(i + 1, nv, nsem)
        pltpu.sync_copy(ones_v, deg_sh.at[cv], add=True)
    plsc.subcore_barrier()

    @pl.when(s == 0)
    def _():
        pltpu.sync_copy(deg_sh, deg_v)
        pltpu.sync_copy(deg_v, out_hbm.at[pl.ds(c * N, N)])


def _make_deg_kernel():
    mesh = plsc.VectorSubcoreMesh(core_axis_name="c", subcore_axis_name="s")
    return pl.kernel(
        _deg_body,
        out_type=jax.ShapeDtypeStruct((NC * N,), jnp.float32),
        mesh=mesh,
        scratch_types=[
            pltpu.VMEM((KD,), jnp.int32),
            pltpu.VMEM((KD,), jnp.int32),
            pltpu.SemaphoreType.DMA,
            pltpu.SemaphoreType.DMA,
            pltpu.VMEM((KD,), jnp.float32),
            pltpu.VMEM((N,), jnp.float32),
            pltpu.VMEM_SHARED((N,), jnp.float32),
        ],
    )


def _scatter_body(g_hbm, row_hbm, col_hbm, z_hbm, out_hbm,
                  ir0, ic0, r0, gs0, ir1, ic1, is0, is1, acc_sh):
    c = lax.axis_index("c")
    s = lax.axis_index("s")
    w = c * NS + s
    nbase = s * RW

    pltpu.sync_copy(z_hbm.at[pl.ds(nbase, RW)], acc_sh.at[pl.ds(nbase, RW)])

    @pl.when(s == 0)
    def _():
        pltpu.sync_copy(z_hbm.at[pl.ds(NS * RW, RTAIL)],
                        acc_sh.at[pl.ds(NS * RW, RTAIL)])

    ebase = w * EWP

    def idx_load(j, ir, ic, isem):
        off = ebase + j * K
        pltpu.async_copy(row_hbm.at[pl.ds(off, K)], ir, isem)
        pltpu.async_copy(col_hbm.at[pl.ds(off, K)], ic, isem)

    def idx_wait(j, ir, ic, isem):
        off = ebase + j * K
        pltpu.make_async_copy(row_hbm.at[pl.ds(off, K)], ir, isem).wait()
        pltpu.make_async_copy(col_hbm.at[pl.ds(off, K)], ic, isem).wait()

    def work(ir, ic):
        pltpu.async_copy(g_hbm.at[ir], r0, gs0).wait()
        pltpu.sync_copy(r0, acc_sh.at[ic], add=True)

    idx_load(0, ir0, ic0, is0)

    def pair(p, carry):
        j0 = 2 * p
        idx_wait(j0, ir0, ic0, is0)
        idx_load(j0 + 1, ir1, ic1, is1)
        work(ir0, ic0)
        idx_wait(j0 + 1, ir1, ic1, is1)
        idx_load(j0 + 2, ir0, ic0, is0)
        work(ir1, ic1)
        return carry

    lax.fori_loop(0, NCHUNK // 2 - 1, pair, 0)

    idx_wait(NCHUNK - 2, ir0, ic0, is0)
    idx_load(NCHUNK - 1, ir1, ic1, is1)
    work(ir0, ic0)
    idx_wait(NCHUNK - 1, ir1, ic1, is1)
    work(ir1, ic1)
    plsc.subcore_barrier()

    pltpu.sync_copy(acc_sh.at[pl.ds(nbase, RW)],
                    out_hbm.at[c, pl.ds(nbase, RW)])

    @pl.when(s == 0)
    def _():
        pltpu.sync_copy(acc_sh.at[pl.ds(NS * RW, RTAIL)],
                        out_hbm.at[c, pl.ds(NS * RW, RTAIL)])


def _make_scatter_kernel():
    mesh = plsc.VectorSubcoreMesh(core_axis_name="c", subcore_axis_name="s")
    return pl.kernel(
        _scatter_body,
        out_type=jax.ShapeDtypeStruct((NC, N, D), jnp.float32),
        mesh=mesh,
        scratch_types=[
            pltpu.VMEM((K,), jnp.int32),
            pltpu.VMEM((K,), jnp.int32),
            pltpu.VMEM((K, D), jnp.float32),
            pltpu.SemaphoreType.DMA,
            pltpu.VMEM((K,), jnp.int32),
            pltpu.VMEM((K,), jnp.int32),
            pltpu.SemaphoreType.DMA,
            pltpu.SemaphoreType.DMA,
            pltpu.VMEM_SHARED((NA, D), jnp.float32),
        ],
    )


def _dotT(a, w):
    return lax.dot_general(a, w, (((1,), (1,)), ((), ())),
                           preferred_element_type=jnp.float32)


def _pre_body(x_ref, w_ref, b_ref, deg_ref, h_ref, g_ref, dis_ref):
    degt = deg_ref[0] + deg_ref[1]
    dis = jnp.where(degt > 0.0, lax.rsqrt(jnp.maximum(degt, 1e-30)), 0.0)
    h = _dotT(x_ref[...], w_ref[...]) + b_ref[...]
    h_ref[...] = h
    g_ref[...] = h * dis
    dis_ref[...] = dis


def _conv_body(s_ref, h_ref, dis_ref, wa_ref, wb_ref, h2_ref, g2_ref):
    dis = dis_ref[...]
    hi = (s_ref[0] + s_ref[1]) * dis
    t = _dotT(hi, wa_ref[...]) + _dotT(h_ref[...], wb_ref[...])
    h2 = jax.nn.gelu(t)
    h2_ref[...] = h2
    g2_ref[...] = h2 * dis


def _final_body(s_ref, h_ref, dis_ref, wa_ref, wb_ref, wc_ref, bc_ref,
                out_ref):
    dis = dis_ref[...]
    hi = (s_ref[0] + s_ref[1]) * dis
    t = _dotT(hi, wa_ref[...]) + _dotT(h_ref[...], wb_ref[...])
    h3 = jax.nn.gelu(t)
    out_ref[...] = _dotT(h3, wc_ref[...]) + bc_ref[...]


_row_spec = pl.BlockSpec((BL, D), lambda i: (i, 0))
_dis_spec = pl.BlockSpec((BL, 1), lambda i: (i, 0))
_s_spec = pl.BlockSpec((NC, BL, D), lambda i: (0, i, 0))
_w_spec = pl.BlockSpec((D, D), lambda i: (0, 0))


def _pre_call(x, w_pre, b_pre2, deg):
    return pl.pallas_call(
        _pre_body,
        grid=(GRID,),
        in_specs=[
            _row_spec,
            _w_spec,
            pl.BlockSpec((1, D), lambda i: (0, 0)),
            pl.BlockSpec((NC, BL, 1), lambda i: (0, i, 0)),
        ],
        out_specs=[_row_spec, _row_spec, _dis_spec],
        out_shape=[
            jax.ShapeDtypeStruct((N, D), jnp.float32),
            jax.ShapeDtypeStruct((N, D), jnp.float32),
            jax.ShapeDtypeStruct((N, 1), jnp.float32),
        ],
    )(x, w_pre, b_pre2, deg)


def _conv_call(s, h, dis, wa, wb):
    return pl.pallas_call(
        _conv_body,
        grid=(GRID,),
        in_specs=[_s_spec, _row_spec, _dis_spec, _w_spec, _w_spec],
        out_specs=[_row_spec, _row_spec],
        out_shape=[
            jax.ShapeDtypeStruct((N, D), jnp.float32),
            jax.ShapeDtypeStruct((N, D), jnp.float32),
        ],
    )(s, h, dis, wa, wb)


def _final_call(s, h, dis, wa, wb, w_cls, b_cls2):
    return pl.pallas_call(
        _final_body,
        grid=(GRID,),
        in_specs=[
            _s_spec, _row_spec, _dis_spec, _w_spec, _w_spec,
            pl.BlockSpec((C, D), lambda i: (0, 0)),
            pl.BlockSpec((1, C), lambda i: (0, 0)),
        ],
        out_specs=pl.BlockSpec((BL, C), lambda i: (i, 0)),
        out_shape=jax.ShapeDtypeStruct((N, C), jnp.float32),
    )(s, h, dis, wa, wb, w_cls, b_cls2)


def kernel(x, edge_index, W_pre, b_pre, W1, W2, W_cls, b_cls):
    row = edge_index[0]
    col = edge_index[1]

    zeros_big = jnp.zeros((N, D), jnp.float32)
    zeros_deg = jnp.zeros((N,), jnp.float32)
    ones_deg = jnp.ones((KD,), jnp.float32)

    deg = _make_deg_kernel()(col, zeros_deg, ones_deg).reshape(NC, N, 1)

    b_pre2 = b_pre.reshape(1, D)
    h, g1, dis = _pre_call(x, W_pre, b_pre2, deg)


    scatter = _make_scatter_kernel()
    s1 = scatter(g1, row, col, zeros_big)

    h2, g2 = _conv_call(s1, h, dis, W1[:, :D], W1[:, D:])

    s2 = scatter(g2, row, col, zeros_big)

    logits = _final_call(s2, h2, dis, W2[:, :D], W2[:, D:],
                         W_cls, b_cls.reshape(1, C))
    return logits
